# baseline (speedup 1.0000x reference)
import jax
import jax.numpy as jnp
from jax.experimental import pallas as pl
from jax.experimental.pallas import tpu as pltpu

jax.config.update("jax_default_matmul_precision", "highest")

_N_CB = 4
_BETA = 0.25
_BM = 256


def _rq_kernel(x_ref, e0, e1, e2, e3, t0, t1, t2, t3,
               qx_ref, ind_ref, loss_ref, unused_ref,
               counts_scr, ee_scr, sumsq_scr):
    i = pl.program_id(0)
    nb = pl.num_programs(0)
    bm, d = x_ref.shape
    k = e0.shape[0]
    e_refs = (e0, e1, e2, e3)
    t_refs = (t0, t1, t2, t3)

    @pl.when(i == 0)
    def _init():
        counts_scr[...] = jnp.zeros_like(counts_scr)
        sumsq_scr[0] = 0.0
        for lvl in range(_N_CB):
            et = t_refs[lvl][...]
            ee_scr[lvl, :] = jnp.sum(et * et, axis=0)

    x = x_ref[...]
    iota = jax.lax.broadcasted_iota(jnp.int32, (bm, k), 1)
    qx = jnp.zeros_like(x)
    sumsq = sumsq_scr[0]
    xx = jnp.sum(x * x, axis=1, keepdims=True)
    for lvl in range(_N_CB):
        xy = jax.lax.dot_general(
            x, t_refs[lvl][...], (((1,), (0,)), ((), ())),
            precision=jax.lax.Precision.HIGHEST,
            preferred_element_type=jnp.float32)
        dist = (xx - 2.0 * xy) + ee_scr[lvl, :][None, :]
        ind = jnp.argmin(dist, axis=1).astype(jnp.int32)
        onehot = (iota == ind[:, None]).astype(jnp.float32)
        counts_scr[lvl, :] += jnp.sum(onehot, axis=0)
        x_q = jax.lax.dot_general(
            onehot, e_refs[lvl][...], (((1,), (0,)), ((), ())),
            precision=jax.lax.Precision.HIGHEST,
            preferred_element_type=jnp.float32)
        qx = qx + x_q
        x = x - x_q
        xx = jnp.sum(x * x, axis=1, keepdims=True)
        sumsq = sumsq + jnp.sum(xx)
        ind_ref[lvl, :] = ind
    qx_ref[...] = qx
    sumsq_scr[0] = sumsq

    @pl.when(i == nb - 1)
    def _fin():
        total = nb * bm * d
        loss_ref[...] = jnp.reshape(
            sumsq_scr[0] * (_BETA / (_N_CB * total)), (1, 1))
        unused_ref[...] = jnp.sum((counts_scr[...] == 0.0).astype(jnp.int32),
                                  keepdims=True)


def kernel(x, embed0, embed1, embed2, embed3):
    b, d = x.shape
    embeds = (embed0, embed1, embed2, embed3)
    k = embed0.shape[0]
    ets = tuple(e.T for e in embeds)
    nb = b // _BM

    out_shapes = (
        jax.ShapeDtypeStruct((b, d), jnp.float32),
        jax.ShapeDtypeStruct((_N_CB, b), jnp.int32),
        jax.ShapeDtypeStruct((1, 1), jnp.float32),
        jax.ShapeDtypeStruct((1, 1), jnp.int32),
    )
    in_specs = (
        [pl.BlockSpec((_BM, d), lambda i: (i, 0))]
        + [pl.BlockSpec((k, d), lambda i: (0, 0)) for _ in range(_N_CB)]
        + [pl.BlockSpec((d, k), lambda i: (0, 0)) for _ in range(_N_CB)]
    )
    out_specs = (
        pl.BlockSpec((_BM, d), lambda i: (i, 0)),
        pl.BlockSpec((_N_CB, _BM), lambda i: (0, i)),
        pl.BlockSpec((1, 1), lambda i: (0, 0)),
        pl.BlockSpec((1, 1), lambda i: (0, 0)),
    )
    qx, inds, loss, unused = pl.pallas_call(
        _rq_kernel,
        grid=(nb,),
        in_specs=in_specs,
        out_specs=out_specs,
        out_shape=out_shapes,
        scratch_shapes=[
            pltpu.VMEM((_N_CB, k), jnp.float32),
            pltpu.VMEM((_N_CB, k), jnp.float32),
            pltpu.SMEM((1,), jnp.float32),
        ],
    )(x, *embeds, *ets)
    output = inds.T.astype(jnp.int64)
    return qx, loss[0, 0], unused[0, 0], output
